# baseline (speedup 1.0000x reference)
import functools

import jax
import jax.numpy as jnp
from jax import lax
from jax.experimental import pallas as pl
from jax.experimental.pallas import tpu as pltpu
from jax.experimental.pallas import tpu_sc as plsc

_NC = 1
_NS = 16
_NW = _NC * _NS

_B = 1024
_B_PER_W = _B // _NW
_CATEGS = 100000
_MAIN = 99328
_TAIL = _CATEGS - _MAIN
_L = 16


@functools.partial(
    pl.kernel,
    out_type=jax.ShapeDtypeStruct((_B,), jnp.float32),
    mesh=plsc.VectorSubcoreMesh(core_axis_name="c", subcore_axis_name="s", num_cores=_NC, num_subcores=_NS),
    scratch_types=[
        pltpu.VMEM((_B_PER_W,), jnp.int32),
        pltpu.VMEM((_B_PER_W,), jnp.int32),
        pltpu.VMEM((_B_PER_W,), jnp.int32),
        pltpu.VMEM((_B_PER_W,), jnp.float32),
        pltpu.VMEM((_B_PER_W,), jnp.float32),
    ],
)
def _gather_kernel(main_hbm, tail_hbm, idx_hbm, out_hbm,
                   idx_v, idx_main_v, idx_tail_v, vals_main_v, vals_tail_v):
    wid = lax.axis_index("s")
    base = wid * _B_PER_W
    pltpu.sync_copy(idx_hbm.at[pl.ds(base, _B_PER_W)], idx_v)
    for j in range(_B_PER_W // _L):
        v = idx_v[pl.ds(j * _L, _L)]
        idx_main_v[pl.ds(j * _L, _L)] = jnp.minimum(v, _MAIN - 1)
        idx_tail_v[pl.ds(j * _L, _L)] = jnp.maximum(v - _MAIN, 0)
    pltpu.sync_copy(main_hbm.at[idx_main_v], vals_main_v)
    pltpu.sync_copy(tail_hbm.at[idx_tail_v], vals_tail_v)
    for j in range(_B_PER_W // _L):
        v = idx_v[pl.ds(j * _L, _L)]
        picked = jnp.where(v >= _MAIN,
                           vals_tail_v[pl.ds(j * _L, _L)],
                           vals_main_v[pl.ds(j * _L, _L)])
        vals_main_v[pl.ds(j * _L, _L)] = picked
    pltpu.sync_copy(vals_main_v, out_hbm.at[pl.ds(base, _B_PER_W)])


def kernel(inputs, categ_bias):
    idx = inputs.reshape(_B).astype(jnp.int32)
    main = lax.slice(categ_bias, (0, 0), (_MAIN, 1)).reshape(_MAIN)
    tail = lax.slice(categ_bias, (_MAIN, 0), (_CATEGS, 1)).reshape(_TAIL)
    return _gather_kernel(main, tail, idx)[:, None]


# speedup vs baseline: 1.2887x; 1.2887x over previous
import functools

import jax
import jax.numpy as jnp
from jax import lax
from jax.experimental import pallas as pl
from jax.experimental.pallas import tpu as pltpu
from jax.experimental.pallas import tpu_sc as plsc

_NC = 1
_NS = 16
_NW = _NC * _NS

_B = 1024
_B_PER_W = _B // _NW
_CATEGS = 100000


@functools.partial(
    pl.kernel,
    out_type=jax.ShapeDtypeStruct((_B,), jnp.float32),
    mesh=plsc.VectorSubcoreMesh(core_axis_name="c", subcore_axis_name="s", num_cores=_NC, num_subcores=_NS),
    scratch_types=[
        pltpu.VMEM((_B_PER_W,), jnp.int32),
        pltpu.VMEM((_B_PER_W,), jnp.float32),
        pltpu.SemaphoreType.DMA,
        pltpu.SemaphoreType.DMA,
    ],
)
def _gather_kernel(table_hbm, idx_hbm, out_hbm, idx_v, vals_v, sem_a, sem_b):
    wid = lax.axis_index("s")
    base = wid * _B_PER_W
    half = _B_PER_W // 2
    ia = pltpu.async_copy(idx_hbm.at[pl.ds(base, half)], idx_v.at[pl.ds(0, half)], sem_a)
    ib = pltpu.async_copy(idx_hbm.at[pl.ds(base + half, half)], idx_v.at[pl.ds(half, half)], sem_b)
    ia.wait()
    ga = pltpu.async_copy(table_hbm.at[idx_v.at[pl.ds(0, half)]], vals_v.at[pl.ds(0, half)], sem_a)
    ib.wait()
    gb = pltpu.async_copy(table_hbm.at[idx_v.at[pl.ds(half, half)]], vals_v.at[pl.ds(half, half)], sem_b)
    ga.wait()
    oa = pltpu.async_copy(vals_v.at[pl.ds(0, half)], out_hbm.at[pl.ds(base, half)], sem_a)
    gb.wait()
    ob = pltpu.async_copy(vals_v.at[pl.ds(half, half)], out_hbm.at[pl.ds(base + half, half)], sem_b)
    oa.wait()
    ob.wait()


def kernel(inputs, categ_bias):
    idx = inputs.reshape(_B).astype(jnp.int32)
    table = categ_bias.reshape(_CATEGS)
    return _gather_kernel(table, idx)[:, None]
